# baseline (speedup 1.0000x reference)
import functools

import jax
import jax.numpy as jnp
from jax import lax
from jax.experimental import pallas as pl
from jax.experimental.pallas import tpu as pltpu
from jax.experimental.pallas import tpu_sc as plsc

NC = 2
NS = 16
NW = NC * NS


def _ceil_to(a, b):
    return (a + b - 1) // b * b


def _sc_agg(N, NPAD, CH0, CH1, D, K, NB, LOCAL=False, COLSPLIT=False):
    RPS = NPAD // NS
    RCH = [(o, min(K, RPS - o)) for o in range(0, RPS, K)]
    TPS = N // NS
    CHM = max(CH0, CH1)
    LAG = max(1, NB // 2)
    VMAX = _ceil_to(CHM + NB, NB)

    mesh = plsc.VectorSubcoreMesh(core_axis_name="c", subcore_axis_name="s",
                                  num_cores=NC, num_subcores=NS)

    @functools.partial(
        pl.kernel,
        out_type=jax.ShapeDtypeStruct((NC, NPAD, D), jnp.float32),
        mesh=mesh,
        scratch_types=(
            [pltpu.VMEM((CHM, K), jnp.int32),
             pltpu.VMEM((CHM, K), jnp.int32)]
            + [pltpu.VMEM((K, D), jnp.float32) for _ in range(NB)]
            + [pltpu.SemaphoreType.DMA for _ in range(2 * NB)]
            + [pltpu.VMEM_SHARED((NPAD, D), jnp.float32)]
            + ([pltpu.VMEM_SHARED((N, D), jnp.float32)] if LOCAL else [])
        ),
        compiler_params=pltpu.CompilerParams(use_tc_tiling_on_sc=False),
    )
    def agg(table, src2, dst2, out, *scr):
        isrc, idst = scr[0], scr[1]
        bufs = scr[2:2 + NB]
        gsem = scr[2 + NB:2 + 2 * NB]
        ssem = scr[2 + 2 * NB:2 + 3 * NB]
        acc = scr[2 + 3 * NB]
        tbl = scr[2 + 3 * NB + 1] if LOCAL else table
        c = lax.axis_index("c")
        s = lax.axis_index("s")
        ch = jnp.where(c == 0, CH0, CH1)
        zeros16 = jnp.zeros((16,), jnp.float32)
        r0 = bufs[0]

        def zrow(r, carry):
            for j in range(D // 16):
                r0[r, pl.ds(j * 16, 16)] = zeros16
            return carry

        lax.fori_loop(0, K, zrow, 0)
        sub0 = s * RPS
        for o, n in RCH:
            pltpu.sync_copy(r0.at[pl.ds(0, n)], acc.at[pl.ds(sub0 + o, n)])

        if LOCAL:
            t0 = s * TPS
            if COLSPLIT:
                pltpu.sync_copy(table.at[c, pl.ds(t0, TPS)],
                                tbl.at[pl.ds(t0, TPS)])
            else:
                pltpu.sync_copy(table.at[pl.ds(t0, TPS)],
                                tbl.at[pl.ds(t0, TPS)])

        if COLSPLIT:
            pltpu.sync_copy(src2.at[pl.ds(s * CH0, CH0)],
                            isrc.at[pl.ds(0, CH0)])
            pltpu.sync_copy(dst2.at[pl.ds(s * CH0, CH0)],
                            idst.at[pl.ds(0, CH0)])
        else:
            @pl.when(c == 0)
            def _():
                pltpu.sync_copy(src2.at[pl.ds(s * CH0, CH0)],
                                isrc.at[pl.ds(0, CH0)])
                pltpu.sync_copy(dst2.at[pl.ds(s * CH0, CH0)],
                                idst.at[pl.ds(0, CH0)])

            @pl.when(c == 1)
            def _():
                pltpu.sync_copy(src2.at[pl.ds(NS * CH0 + s * CH1, CH1)],
                                isrc.at[pl.ds(0, CH1)])
                pltpu.sync_copy(dst2.at[pl.ds(NS * CH0 + s * CH1, CH1)],
                                idst.at[pl.ds(0, CH1)])

        plsc.subcore_barrier()

        def gstart(i, b):
            pltpu.async_copy(tbl.at[isrc.at[i]], bufs[b], gsem[b])

        def gwait(i, b):
            pltpu.make_async_copy(tbl.at[isrc.at[i]], bufs[b],
                                  gsem[b]).wait()

        def sstart(i, b):
            pltpu.async_copy(bufs[b], acc.at[idst.at[i]], ssem[b], add=True)

        def swait(i, b):
            pltpu.make_async_copy(bufs[b], acc.at[idst.at[i]],
                                  ssem[b]).wait()

        def step(j, carry):
            for b in range(NB):
                v = j * NB + b

                @pl.when((v >= NB) & (v - NB < ch))
                def _():
                    swait(v - NB, b)

                @pl.when(v < ch)
                def _():
                    gstart(v, b)

                u = v - LAG
                bu = (b - LAG) % NB

                @pl.when((u >= 0) & (u < ch))
                def _():
                    gwait(u, bu)
                    sstart(u, bu)

            return carry

        lax.fori_loop(0, VMAX // NB, step, 0)

        plsc.subcore_barrier()
        for o, n in RCH:
            sl = pl.ds(sub0 + o, n)
            pltpu.sync_copy(acc.at[sl], out.at[c, sl])

    return agg


def _sc_deg(NPAD, CH0, CH1, K):
    D = 16
    RPS = NPAD // NS
    RCH = [(o, min(K, RPS - o)) for o in range(0, RPS, K)]
    CHM = max(CH0, CH1)
    FD = 8

    mesh = plsc.VectorSubcoreMesh(core_axis_name="c", subcore_axis_name="s",
                                  num_cores=NC, num_subcores=NS)

    @functools.partial(
        pl.kernel,
        out_type=jax.ShapeDtypeStruct((NC, NPAD, D), jnp.float32),
        mesh=mesh,
        scratch_types=[
            pltpu.VMEM((CHM, K), jnp.int32),
            pltpu.VMEM((K, D), jnp.float32),
            pltpu.VMEM_SHARED((NPAD, D), jnp.float32),
            pltpu.SemaphoreType.DMA,
        ],
        compiler_params=pltpu.CompilerParams(use_tc_tiling_on_sc=False),
    )
    def deg(dst2, out, idst, rows, acc, sem):
        c = lax.axis_index("c")
        s = lax.axis_index("s")
        ch = jnp.where(c == 0, CH0, CH1)
        zeros16 = jnp.zeros((16,), jnp.float32)
        ones16 = jnp.ones((16,), jnp.float32)

        def fill(vec):
            def body(r, carry):
                rows[r, pl.ds(0, 16)] = vec
                return carry
            lax.fori_loop(0, K, body, 0)

        fill(zeros16)
        sub0 = s * RPS
        for o, n in RCH:
            pltpu.sync_copy(rows.at[pl.ds(0, n)], acc.at[pl.ds(sub0 + o, n)])
        fill(ones16)

        @pl.when(c == 0)
        def _():
            pltpu.sync_copy(dst2.at[pl.ds(s * CH0, CH0)],
                            idst.at[pl.ds(0, CH0)])

        @pl.when(c == 1)
        def _():
            pltpu.sync_copy(dst2.at[pl.ds(NS * CH0 + s * CH1, CH1)],
                            idst.at[pl.ds(0, CH1)])

        plsc.subcore_barrier()

        def grp(g, carry):
            for t in range(FD):
                i = g * FD + t

                @pl.when(i < ch)
                def _():
                    pltpu.async_copy(rows, acc.at[idst.at[i]], sem, add=True)

            for t in range(FD):
                i = g * FD + t

                @pl.when(i < ch)
                def _():
                    pltpu.make_async_copy(rows, acc.at[idst.at[i]],
                                          sem).wait()

            return carry

        lax.fori_loop(0, (CHM + FD - 1) // FD, grp, 0)
        plsc.subcore_barrier()
        for o, n in RCH:
            sl = pl.ds(sub0 + o, n)
            pltpu.sync_copy(acc.at[sl], out.at[c, sl])

    return deg


def _pre_body(cnt_ref, x_ref, dis_ref, xs_ref):
    c = cnt_ref[0, :, 0:1] + cnt_ref[1, :, 0:1] + 1.0
    d = lax.rsqrt(c)
    dis_ref[...] = d
    h = x_ref.shape[1] // 2
    xs_ref[0] = d * x_ref[:, :h]
    xs_ref[1] = d * x_ref[:, h:]


def _l1_body(dis_ref, xs_ref, agg_ref, w1_ref, b1_ref, w2_ref, ys2_ref):
    d = dis_ref[...]
    sfull = jnp.concatenate(
        [agg_ref[0] + xs_ref[0], agg_ref[1] + xs_ref[1]], axis=1)
    a = d * sfull
    h1 = jnp.maximum(
        jnp.dot(a, w1_ref[...], preferred_element_type=jnp.float32)
        + b1_ref[...], 0.0)
    ys2_ref[...] = d * jnp.dot(h1, w2_ref[...],
                               preferred_element_type=jnp.float32)


def _l2_body(dis_ref, ys2_ref, agg_ref, b2_ref, w3_ref, ys3_ref):
    d = dis_ref[...]
    h2 = jnp.maximum(
        d * (agg_ref[0] + agg_ref[1] + ys2_ref[...]) + b2_ref[...], 0.0)
    ys3_ref[...] = d * jnp.dot(h2, w3_ref[...],
                               preferred_element_type=jnp.float32)


def _l3_body(dis_ref, ys3_ref, agg_ref, b3_ref, out_ref):
    d = dis_ref[...]
    o = d * (agg_ref[0] + agg_ref[1] + ys3_ref[...]) + b3_ref[...]
    lg = o[:, :10]
    m = jnp.max(lg, axis=1, keepdims=True)
    e = jnp.exp(lg - m)
    res = lg - m - jnp.log(jnp.sum(e, axis=1, keepdims=True))
    out_ref[...] = jnp.concatenate(
        [res, jnp.zeros((res.shape[0], 6), jnp.float32)], axis=1)


def kernel(x, edge_index, W1, b1, W2, b2, W3, b3):
    N, D_IN = x.shape
    E = edge_index.shape[1]
    H1 = W1.shape[1]
    H2 = W2.shape[1]
    C = W3.shape[1]
    CP = 16

    K1 = 32
    K2 = 32
    NPAD = _ceil_to(N + 1, NS * 8)

    src = edge_index[0]
    dst = edge_index[1]

    def _chunked(K, F0):
        tot = -(-E // (NS * K))
        ch0 = max(2, int(round(F0 * tot / 2)) * 2)
        ch1 = max(2, -(-(tot - ch0) // 2) * 2)
        ep = NS * K * (ch0 + ch1)
        s, d = src, dst
        if ep != E:
            pad = ep - E
            s = jnp.concatenate([s, jnp.zeros((pad,), s.dtype)])
            d = jnp.concatenate([d, jnp.full((pad,), N, d.dtype)])
        return ch0, ch1, s.reshape(ep // K, K), d.reshape(ep // K, K)

    def _chunked_full(K):
        tot = -(-E // (NS * K))
        ch = (tot + 1) // 2 * 2
        ep = NS * K * ch
        s, d = src, dst
        if ep != E:
            pad = ep - E
            s = jnp.concatenate([s, jnp.zeros((pad,), s.dtype)])
            d = jnp.concatenate([d, jnp.full((pad,), N, d.dtype)])
        return ch, s.reshape(ep // K, K), d.reshape(ep // K, K)

    AC, src2a, dst2a = _chunked_full(K1)
    B0, B1, src2b, dst2b = _chunked(K2, 0.50)

    W3p = jnp.pad(W3, ((0, 0), (0, CP - C)))
    b1r = b1.reshape(1, H1)
    b2r = b2.reshape(1, H2)
    b3r = jnp.pad(b3, (0, CP - C)).reshape(1, CP)

    BN = 2000
    G = N // BN
    f32 = jnp.float32

    cnt = _sc_deg(NPAD, B0, B1, K2)(dst2b)

    DH = D_IN // 2
    dis, xs = pl.pallas_call(
        _pre_body,
        grid=(G,),
        in_specs=[
            pl.BlockSpec((NC, BN, 16), lambda i: (0, i, 0)),
            pl.BlockSpec((BN, D_IN), lambda i: (i, 0)),
        ],
        out_specs=[
            pl.BlockSpec((BN, 1), lambda i: (i, 0)),
            pl.BlockSpec((NC, BN, DH), lambda i: (0, i, 0)),
        ],
        out_shape=[
            jax.ShapeDtypeStruct((N, 1), f32),
            jax.ShapeDtypeStruct((NC, N, DH), f32),
        ],
    )(cnt, x)

    agg1 = _sc_agg(N, NPAD, AC, AC, DH, K1, 5, LOCAL=True,
                   COLSPLIT=True)(xs, src2a, dst2a)

    ys2 = pl.pallas_call(
        _l1_body,
        grid=(G,),
        in_specs=[
            pl.BlockSpec((BN, 1), lambda i: (i, 0)),
            pl.BlockSpec((NC, BN, DH), lambda i: (0, i, 0)),
            pl.BlockSpec((NC, BN, DH), lambda i: (0, i, 0)),
            pl.BlockSpec((D_IN, H1), lambda i: (0, 0)),
            pl.BlockSpec((1, H1), lambda i: (0, 0)),
            pl.BlockSpec((H1, H2), lambda i: (0, 0)),
        ],
        out_specs=pl.BlockSpec((BN, H2), lambda i: (i, 0)),
        out_shape=jax.ShapeDtypeStruct((N, H2), f32),
    )(dis, xs, agg1, W1, b1r, W2)

    agg2 = _sc_agg(N, NPAD, B0, B1, H2, K2, 8, LOCAL=True)(ys2, src2b, dst2b)

    ys3 = pl.pallas_call(
        _l2_body,
        grid=(G,),
        in_specs=[
            pl.BlockSpec((BN, 1), lambda i: (i, 0)),
            pl.BlockSpec((BN, H2), lambda i: (i, 0)),
            pl.BlockSpec((NC, BN, H2), lambda i: (0, i, 0)),
            pl.BlockSpec((1, H2), lambda i: (0, 0)),
            pl.BlockSpec((H2, CP), lambda i: (0, 0)),
        ],
        out_specs=pl.BlockSpec((BN, CP), lambda i: (i, 0)),
        out_shape=jax.ShapeDtypeStruct((N, CP), f32),
    )(dis, ys2, agg2, b2r, W3p)

    agg3 = _sc_agg(N, NPAD, B0, B1, CP, K2, 8, LOCAL=True)(ys3, src2b, dst2b)

    outp = pl.pallas_call(
        _l3_body,
        grid=(G,),
        in_specs=[
            pl.BlockSpec((BN, 1), lambda i: (i, 0)),
            pl.BlockSpec((BN, CP), lambda i: (i, 0)),
            pl.BlockSpec((NC, BN, CP), lambda i: (0, i, 0)),
            pl.BlockSpec((1, CP), lambda i: (0, 0)),
        ],
        out_specs=pl.BlockSpec((BN, CP), lambda i: (i, 0)),
        out_shape=jax.ShapeDtypeStruct((N, CP), f32),
    )(dis, ys3, agg3, b3r)

    return outp[:, :C]


# speedup vs baseline: 1.0377x; 1.0377x over previous
import functools

import jax
import jax.numpy as jnp
from jax import lax
from jax.experimental import pallas as pl
from jax.experimental.pallas import tpu as pltpu
from jax.experimental.pallas import tpu_sc as plsc

NC = 2
NS = 16
NW = NC * NS


def _ceil_to(a, b):
    return (a + b - 1) // b * b


def _sc_agg(N, NPAD, CH0, CH1, D, K, NB, LOCAL=False, COLSPLIT=False):
    RPS = NPAD // NS
    RCH = [(o, min(K, RPS - o)) for o in range(0, RPS, K)]
    TPS = N // NS
    CHM = max(CH0, CH1)
    LAG = max(1, NB // 2)
    VMAX = _ceil_to(CHM + NB, NB)

    mesh = plsc.VectorSubcoreMesh(core_axis_name="c", subcore_axis_name="s",
                                  num_cores=NC, num_subcores=NS)

    @functools.partial(
        pl.kernel,
        out_type=jax.ShapeDtypeStruct((NC, NPAD, D), jnp.float32),
        mesh=mesh,
        scratch_types=(
            [pltpu.VMEM((CHM, K), jnp.int32),
             pltpu.VMEM((CHM, K), jnp.int32)]
            + [pltpu.VMEM((K, D), jnp.float32) for _ in range(NB)]
            + [pltpu.SemaphoreType.DMA for _ in range(2 * NB)]
            + [pltpu.VMEM_SHARED((NPAD, D), jnp.float32)]
            + ([pltpu.VMEM_SHARED((N, D), jnp.float32)] if LOCAL else [])
        ),
        compiler_params=pltpu.CompilerParams(use_tc_tiling_on_sc=False),
    )
    def agg(table, src2, dst2, out, *scr):
        isrc, idst = scr[0], scr[1]
        bufs = scr[2:2 + NB]
        gsem = scr[2 + NB:2 + 2 * NB]
        ssem = scr[2 + 2 * NB:2 + 3 * NB]
        acc = scr[2 + 3 * NB]
        tbl = scr[2 + 3 * NB + 1] if LOCAL else table
        c = lax.axis_index("c")
        s = lax.axis_index("s")
        ch = jnp.where(c == 0, CH0, CH1)
        zeros16 = jnp.zeros((16,), jnp.float32)
        r0 = bufs[0]

        def zrow(r, carry):
            for j in range(D // 16):
                r0[r, pl.ds(j * 16, 16)] = zeros16
            return carry

        lax.fori_loop(0, K, zrow, 0)
        sub0 = s * RPS
        for o, n in RCH:
            pltpu.sync_copy(r0.at[pl.ds(0, n)], acc.at[pl.ds(sub0 + o, n)])

        if LOCAL:
            t0 = s * TPS
            if COLSPLIT:
                pltpu.sync_copy(table.at[c, pl.ds(t0, TPS)],
                                tbl.at[pl.ds(t0, TPS)])
            else:
                pltpu.sync_copy(table.at[pl.ds(t0, TPS)],
                                tbl.at[pl.ds(t0, TPS)])

        if COLSPLIT:
            pltpu.sync_copy(src2.at[pl.ds(s * CH0, CH0)],
                            isrc.at[pl.ds(0, CH0)])
            pltpu.sync_copy(dst2.at[pl.ds(s * CH0, CH0)],
                            idst.at[pl.ds(0, CH0)])
        else:
            @pl.when(c == 0)
            def _():
                pltpu.sync_copy(src2.at[pl.ds(s * CH0, CH0)],
                                isrc.at[pl.ds(0, CH0)])
                pltpu.sync_copy(dst2.at[pl.ds(s * CH0, CH0)],
                                idst.at[pl.ds(0, CH0)])

            @pl.when(c == 1)
            def _():
                pltpu.sync_copy(src2.at[pl.ds(NS * CH0 + s * CH1, CH1)],
                                isrc.at[pl.ds(0, CH1)])
                pltpu.sync_copy(dst2.at[pl.ds(NS * CH0 + s * CH1, CH1)],
                                idst.at[pl.ds(0, CH1)])

        plsc.subcore_barrier()

        def gstart(i, b):
            pltpu.async_copy(tbl.at[isrc.at[i]], bufs[b], gsem[b])

        def gwait(i, b):
            pltpu.make_async_copy(tbl.at[isrc.at[i]], bufs[b],
                                  gsem[b]).wait()

        def sstart(i, b):
            pltpu.async_copy(bufs[b], acc.at[idst.at[i]], ssem[b], add=True)

        def swait(i, b):
            pltpu.make_async_copy(bufs[b], acc.at[idst.at[i]],
                                  ssem[b]).wait()

        def step(j, carry):
            for b in range(NB):
                v = j * NB + b

                @pl.when((v >= NB) & (v - NB < ch))
                def _():
                    swait(v - NB, b)

                @pl.when(v < ch)
                def _():
                    gstart(v, b)

                u = v - LAG
                bu = (b - LAG) % NB

                @pl.when((u >= 0) & (u < ch))
                def _():
                    gwait(u, bu)
                    sstart(u, bu)

            return carry

        lax.fori_loop(0, VMAX // NB, step, 0)

        plsc.subcore_barrier()
        for o, n in RCH:
            sl = pl.ds(sub0 + o, n)
            pltpu.sync_copy(acc.at[sl], out.at[c, sl])

    return agg


def _sc_deg(NPAD, CH0, CH1, K):
    D = 16
    RPS = NPAD // NS
    RCH = [(o, min(K, RPS - o)) for o in range(0, RPS, K)]
    CHM = max(CH0, CH1)
    FD = 8

    mesh = plsc.VectorSubcoreMesh(core_axis_name="c", subcore_axis_name="s",
                                  num_cores=NC, num_subcores=NS)

    @functools.partial(
        pl.kernel,
        out_type=jax.ShapeDtypeStruct((NC, NPAD, D), jnp.float32),
        mesh=mesh,
        scratch_types=[
            pltpu.VMEM((CHM, K), jnp.int32),
            pltpu.VMEM((K, D), jnp.float32),
            pltpu.VMEM_SHARED((NPAD, D), jnp.float32),
            pltpu.SemaphoreType.DMA,
        ],
        compiler_params=pltpu.CompilerParams(use_tc_tiling_on_sc=False),
    )
    def deg(dst2, out, idst, rows, acc, sem):
        c = lax.axis_index("c")
        s = lax.axis_index("s")
        ch = jnp.where(c == 0, CH0, CH1)
        zeros16 = jnp.zeros((16,), jnp.float32)
        ones16 = jnp.ones((16,), jnp.float32)

        def fill(vec):
            def body(r, carry):
                rows[r, pl.ds(0, 16)] = vec
                return carry
            lax.fori_loop(0, K, body, 0)

        fill(zeros16)
        sub0 = s * RPS
        for o, n in RCH:
            pltpu.sync_copy(rows.at[pl.ds(0, n)], acc.at[pl.ds(sub0 + o, n)])
        fill(ones16)

        @pl.when(c == 0)
        def _():
            pltpu.sync_copy(dst2.at[pl.ds(s * CH0, CH0)],
                            idst.at[pl.ds(0, CH0)])

        @pl.when(c == 1)
        def _():
            pltpu.sync_copy(dst2.at[pl.ds(NS * CH0 + s * CH1, CH1)],
                            idst.at[pl.ds(0, CH1)])

        plsc.subcore_barrier()

        def grp(g, carry):
            for t in range(FD):
                i = g * FD + t

                @pl.when(i < ch)
                def _():
                    pltpu.async_copy(rows, acc.at[idst.at[i]], sem, add=True)

            for t in range(FD):
                i = g * FD + t

                @pl.when(i < ch)
                def _():
                    pltpu.make_async_copy(rows, acc.at[idst.at[i]],
                                          sem).wait()

            return carry

        lax.fori_loop(0, (CHM + FD - 1) // FD, grp, 0)
        plsc.subcore_barrier()
        for o, n in RCH:
            sl = pl.ds(sub0 + o, n)
            pltpu.sync_copy(acc.at[sl], out.at[c, sl])

    return deg


def _pre_body(cnt_ref, x_ref, dis_ref, xs_ref):
    c = cnt_ref[0, :, 0:1] + cnt_ref[1, :, 0:1] + 1.0
    d = lax.rsqrt(c)
    dis_ref[...] = d
    h = x_ref.shape[1] // 2
    xs_ref[0] = d * x_ref[:, :h]
    xs_ref[1] = d * x_ref[:, h:]


def _l1_body(dis_ref, xs_ref, agg_ref, w1_ref, b1_ref, w2_ref, ys2_ref):
    d = dis_ref[...]
    sfull = jnp.concatenate(
        [agg_ref[0] + xs_ref[0], agg_ref[1] + xs_ref[1]], axis=1)
    a = d * sfull
    h1 = jnp.maximum(
        jnp.dot(a, w1_ref[...], preferred_element_type=jnp.float32)
        + b1_ref[...], 0.0)
    ys2_ref[...] = d * jnp.dot(h1, w2_ref[...],
                               preferred_element_type=jnp.float32)


def _l2_body(dis_ref, ys2_ref, agg_ref, b2_ref, w3_ref, ys3_ref):
    d = dis_ref[...]
    h2 = jnp.maximum(
        d * (agg_ref[0] + agg_ref[1] + ys2_ref[...]) + b2_ref[...], 0.0)
    ys3_ref[...] = d * jnp.dot(h2, w3_ref[...],
                               preferred_element_type=jnp.float32)


def _l3_body(dis_ref, ys3_ref, agg_ref, b3_ref, out_ref):
    d = dis_ref[...]
    o = d * (agg_ref[0] + agg_ref[1] + ys3_ref[...]) + b3_ref[...]
    lg = o[:, :10]
    m = jnp.max(lg, axis=1, keepdims=True)
    e = jnp.exp(lg - m)
    res = lg - m - jnp.log(jnp.sum(e, axis=1, keepdims=True))
    out_ref[...] = jnp.concatenate(
        [res, jnp.zeros((res.shape[0], 6), jnp.float32)], axis=1)


def kernel(x, edge_index, W1, b1, W2, b2, W3, b3):
    N, D_IN = x.shape
    E = edge_index.shape[1]
    H1 = W1.shape[1]
    H2 = W2.shape[1]
    C = W3.shape[1]
    CP = 16

    K1 = 32
    K2 = 64
    NPAD = _ceil_to(N + 1, NS * 8)

    src = edge_index[0]
    dst = edge_index[1]

    def _chunked(K, F0):
        tot = -(-E // (NS * K))
        ch0 = max(2, int(round(F0 * tot / 2)) * 2)
        ch1 = max(2, -(-(tot - ch0) // 2) * 2)
        ep = NS * K * (ch0 + ch1)
        s, d = src, dst
        if ep != E:
            pad = ep - E
            s = jnp.concatenate([s, jnp.zeros((pad,), s.dtype)])
            d = jnp.concatenate([d, jnp.full((pad,), N, d.dtype)])
        return ch0, ch1, s.reshape(ep // K, K), d.reshape(ep // K, K)

    def _chunked_full(K):
        tot = -(-E // (NS * K))
        ch = (tot + 1) // 2 * 2
        ep = NS * K * ch
        s, d = src, dst
        if ep != E:
            pad = ep - E
            s = jnp.concatenate([s, jnp.zeros((pad,), s.dtype)])
            d = jnp.concatenate([d, jnp.full((pad,), N, d.dtype)])
        return ch, s.reshape(ep // K, K), d.reshape(ep // K, K)

    AC, src2a, dst2a = _chunked_full(K1)
    B0, B1, src2b, dst2b = _chunked(K2, 0.50)

    W3p = jnp.pad(W3, ((0, 0), (0, CP - C)))
    b1r = b1.reshape(1, H1)
    b2r = b2.reshape(1, H2)
    b3r = jnp.pad(b3, (0, CP - C)).reshape(1, CP)

    BN = N
    G = N // BN
    f32 = jnp.float32

    cnt = _sc_deg(NPAD, B0, B1, K2)(dst2b)

    DH = D_IN // 2
    dis, xs = pl.pallas_call(
        _pre_body,
        grid=(G,),
        in_specs=[
            pl.BlockSpec((NC, BN, 16), lambda i: (0, i, 0)),
            pl.BlockSpec((BN, D_IN), lambda i: (i, 0)),
        ],
        out_specs=[
            pl.BlockSpec((BN, 1), lambda i: (i, 0)),
            pl.BlockSpec((NC, BN, DH), lambda i: (0, i, 0)),
        ],
        out_shape=[
            jax.ShapeDtypeStruct((N, 1), f32),
            jax.ShapeDtypeStruct((NC, N, DH), f32),
        ],
    )(cnt, x)

    agg1 = _sc_agg(N, NPAD, AC, AC, DH, K1, 5, LOCAL=True,
                   COLSPLIT=True)(xs, src2a, dst2a)

    ys2 = pl.pallas_call(
        _l1_body,
        grid=(G,),
        in_specs=[
            pl.BlockSpec((BN, 1), lambda i: (i, 0)),
            pl.BlockSpec((NC, BN, DH), lambda i: (0, i, 0)),
            pl.BlockSpec((NC, BN, DH), lambda i: (0, i, 0)),
            pl.BlockSpec((D_IN, H1), lambda i: (0, 0)),
            pl.BlockSpec((1, H1), lambda i: (0, 0)),
            pl.BlockSpec((H1, H2), lambda i: (0, 0)),
        ],
        out_specs=pl.BlockSpec((BN, H2), lambda i: (i, 0)),
        out_shape=jax.ShapeDtypeStruct((N, H2), f32),
    )(dis, xs, agg1, W1, b1r, W2)

    agg2 = _sc_agg(N, NPAD, B0, B1, H2, K2, 6, LOCAL=True)(ys2, src2b, dst2b)

    ys3 = pl.pallas_call(
        _l2_body,
        grid=(G,),
        in_specs=[
            pl.BlockSpec((BN, 1), lambda i: (i, 0)),
            pl.BlockSpec((BN, H2), lambda i: (i, 0)),
            pl.BlockSpec((NC, BN, H2), lambda i: (0, i, 0)),
            pl.BlockSpec((1, H2), lambda i: (0, 0)),
            pl.BlockSpec((H2, CP), lambda i: (0, 0)),
        ],
        out_specs=pl.BlockSpec((BN, CP), lambda i: (i, 0)),
        out_shape=jax.ShapeDtypeStruct((N, CP), f32),
    )(dis, ys2, agg2, b2r, W3p)

    agg3 = _sc_agg(N, NPAD, B0, B1, CP, K2, 6, LOCAL=True)(ys3, src2b, dst2b)

    outp = pl.pallas_call(
        _l3_body,
        grid=(G,),
        in_specs=[
            pl.BlockSpec((BN, 1), lambda i: (i, 0)),
            pl.BlockSpec((BN, CP), lambda i: (i, 0)),
            pl.BlockSpec((NC, BN, CP), lambda i: (0, i, 0)),
            pl.BlockSpec((1, CP), lambda i: (0, 0)),
        ],
        out_specs=pl.BlockSpec((BN, CP), lambda i: (i, 0)),
        out_shape=jax.ShapeDtypeStruct((N, CP), f32),
    )(dis, ys3, agg3, b3r)

    return outp[:, :C]


# speedup vs baseline: 1.0482x; 1.0102x over previous
import functools

import jax
import jax.numpy as jnp
from jax import lax
from jax.experimental import pallas as pl
from jax.experimental.pallas import tpu as pltpu
from jax.experimental.pallas import tpu_sc as plsc

NC = 2
NS = 16
NW = NC * NS


def _ceil_to(a, b):
    return (a + b - 1) // b * b


def _sc_agg(N, NPAD, CH0, CH1, D, K, NB, LOCAL=False, COLSPLIT=False):
    RPS = NPAD // NS
    RCH = [(o, min(K, RPS - o)) for o in range(0, RPS, K)]
    TPS = N // NS
    CHM = max(CH0, CH1)
    LAG = max(1, NB // 2)
    VMAX = _ceil_to(CHM + NB, NB)

    mesh = plsc.VectorSubcoreMesh(core_axis_name="c", subcore_axis_name="s",
                                  num_cores=NC, num_subcores=NS)

    @functools.partial(
        pl.kernel,
        out_type=jax.ShapeDtypeStruct((NC, NPAD, D), jnp.float32),
        mesh=mesh,
        scratch_types=(
            [pltpu.VMEM((CHM, K), jnp.int32),
             pltpu.VMEM((CHM, K), jnp.int32)]
            + [pltpu.VMEM((K, D), jnp.float32) for _ in range(NB)]
            + [pltpu.SemaphoreType.DMA for _ in range(2 * NB)]
            + [pltpu.VMEM_SHARED((NPAD, D), jnp.float32)]
            + ([pltpu.VMEM_SHARED((N, D), jnp.float32)] if LOCAL else [])
        ),
        compiler_params=pltpu.CompilerParams(use_tc_tiling_on_sc=False),
    )
    def agg(table, src2, dst2, out, *scr):
        isrc, idst = scr[0], scr[1]
        bufs = scr[2:2 + NB]
        gsem = scr[2 + NB:2 + 2 * NB]
        ssem = scr[2 + 2 * NB:2 + 3 * NB]
        acc = scr[2 + 3 * NB]
        tbl = scr[2 + 3 * NB + 1] if LOCAL else table
        c = lax.axis_index("c")
        s = lax.axis_index("s")
        ch = jnp.where(c == 0, CH0, CH1)
        zeros16 = jnp.zeros((16,), jnp.float32)
        r0 = bufs[0]

        def zrow(r, carry):
            for j in range(D // 16):
                r0[r, pl.ds(j * 16, 16)] = zeros16
            return carry

        lax.fori_loop(0, K, zrow, 0)
        sub0 = s * RPS
        for o, n in RCH:
            pltpu.sync_copy(r0.at[pl.ds(0, n)], acc.at[pl.ds(sub0 + o, n)])

        if LOCAL:
            t0 = s * TPS
            if COLSPLIT:
                pltpu.sync_copy(table.at[c, pl.ds(t0, TPS)],
                                tbl.at[pl.ds(t0, TPS)])
            else:
                pltpu.sync_copy(table.at[pl.ds(t0, TPS)],
                                tbl.at[pl.ds(t0, TPS)])

        if COLSPLIT:
            pltpu.sync_copy(src2.at[pl.ds(s * CH0, CH0)],
                            isrc.at[pl.ds(0, CH0)])
            pltpu.sync_copy(dst2.at[pl.ds(s * CH0, CH0)],
                            idst.at[pl.ds(0, CH0)])
        else:
            @pl.when(c == 0)
            def _():
                pltpu.sync_copy(src2.at[pl.ds(s * CH0, CH0)],
                                isrc.at[pl.ds(0, CH0)])
                pltpu.sync_copy(dst2.at[pl.ds(s * CH0, CH0)],
                                idst.at[pl.ds(0, CH0)])

            @pl.when(c == 1)
            def _():
                pltpu.sync_copy(src2.at[pl.ds(NS * CH0 + s * CH1, CH1)],
                                isrc.at[pl.ds(0, CH1)])
                pltpu.sync_copy(dst2.at[pl.ds(NS * CH0 + s * CH1, CH1)],
                                idst.at[pl.ds(0, CH1)])

        plsc.subcore_barrier()

        def gstart(i, b):
            pltpu.async_copy(tbl.at[isrc.at[i]], bufs[b], gsem[b])

        def gwait(i, b):
            pltpu.make_async_copy(tbl.at[isrc.at[i]], bufs[b],
                                  gsem[b]).wait()

        def sstart(i, b):
            pltpu.async_copy(bufs[b], acc.at[idst.at[i]], ssem[b], add=True)

        def swait(i, b):
            pltpu.make_async_copy(bufs[b], acc.at[idst.at[i]],
                                  ssem[b]).wait()

        def step(j, carry):
            for b in range(NB):
                v = j * NB + b

                @pl.when((v >= NB) & (v - NB < ch))
                def _():
                    swait(v - NB, b)

                @pl.when(v < ch)
                def _():
                    gstart(v, b)

                u = v - LAG
                bu = (b - LAG) % NB

                @pl.when((u >= 0) & (u < ch))
                def _():
                    gwait(u, bu)
                    sstart(u, bu)

            return carry

        lax.fori_loop(0, VMAX // NB, step, 0)

        plsc.subcore_barrier()
        for o, n in RCH:
            sl = pl.ds(sub0 + o, n)
            pltpu.sync_copy(acc.at[sl], out.at[c, sl])

    return agg


def _sc_deg(NPAD, CH0, CH1, K):
    D = 16
    RPS = NPAD // NS
    RCH = [(o, min(K, RPS - o)) for o in range(0, RPS, K)]
    CHM = max(CH0, CH1)
    FD = 8

    mesh = plsc.VectorSubcoreMesh(core_axis_name="c", subcore_axis_name="s",
                                  num_cores=NC, num_subcores=NS)

    @functools.partial(
        pl.kernel,
        out_type=jax.ShapeDtypeStruct((NC, NPAD, D), jnp.float32),
        mesh=mesh,
        scratch_types=[
            pltpu.VMEM((CHM, K), jnp.int32),
            pltpu.VMEM((K, D), jnp.float32),
            pltpu.VMEM_SHARED((NPAD, D), jnp.float32),
            pltpu.SemaphoreType.DMA,
        ],
        compiler_params=pltpu.CompilerParams(use_tc_tiling_on_sc=False),
    )
    def deg(dst2, out, idst, rows, acc, sem):
        c = lax.axis_index("c")
        s = lax.axis_index("s")
        ch = jnp.where(c == 0, CH0, CH1)
        zeros16 = jnp.zeros((16,), jnp.float32)
        ones16 = jnp.ones((16,), jnp.float32)

        def fill(vec):
            def body(r, carry):
                rows[r, pl.ds(0, 16)] = vec
                return carry
            lax.fori_loop(0, K, body, 0)

        fill(zeros16)
        sub0 = s * RPS
        for o, n in RCH:
            pltpu.sync_copy(rows.at[pl.ds(0, n)], acc.at[pl.ds(sub0 + o, n)])
        fill(ones16)

        @pl.when(c == 0)
        def _():
            pltpu.sync_copy(dst2.at[pl.ds(s * CH0, CH0)],
                            idst.at[pl.ds(0, CH0)])

        @pl.when(c == 1)
        def _():
            pltpu.sync_copy(dst2.at[pl.ds(NS * CH0 + s * CH1, CH1)],
                            idst.at[pl.ds(0, CH1)])

        plsc.subcore_barrier()

        def grp(g, carry):
            for t in range(FD):
                i = g * FD + t

                @pl.when(i < ch)
                def _():
                    pltpu.async_copy(rows, acc.at[idst.at[i]], sem, add=True)

            for t in range(FD):
                i = g * FD + t

                @pl.when(i < ch)
                def _():
                    pltpu.make_async_copy(rows, acc.at[idst.at[i]],
                                          sem).wait()

            return carry

        lax.fori_loop(0, (CHM + FD - 1) // FD, grp, 0)
        plsc.subcore_barrier()
        for o, n in RCH:
            sl = pl.ds(sub0 + o, n)
            pltpu.sync_copy(acc.at[sl], out.at[c, sl])

    return deg


def _pre_body(cnt_ref, x_ref, dis_ref, xs_ref):
    c = cnt_ref[0, :, 0:1] + cnt_ref[1, :, 0:1] + 1.0
    d = lax.rsqrt(c)
    dis_ref[...] = d
    h = x_ref.shape[1] // 2
    xs_ref[0] = d * x_ref[:, :h]
    xs_ref[1] = d * x_ref[:, h:]


def _l1_body(dis_ref, xs_ref, agg_ref, w1_ref, b1_ref, w2_ref, ys2_ref):
    d = dis_ref[...]
    sfull = jnp.concatenate(
        [agg_ref[0] + xs_ref[0], agg_ref[1] + xs_ref[1]], axis=1)
    a = d * sfull
    h1 = jnp.maximum(
        jnp.dot(a, w1_ref[...], preferred_element_type=jnp.float32)
        + b1_ref[...], 0.0)
    ys2_ref[...] = d * jnp.dot(h1, w2_ref[...],
                               preferred_element_type=jnp.float32)


def _l2_body(dis_ref, ys2_ref, agg_ref, b2_ref, w3_ref, ys3_ref):
    d = dis_ref[...]
    h2 = jnp.maximum(
        d * (agg_ref[0] + agg_ref[1] + ys2_ref[...]) + b2_ref[...], 0.0)
    ys3_ref[...] = d * jnp.dot(h2, w3_ref[...],
                               preferred_element_type=jnp.float32)


def _l3_body(dis_ref, ys3_ref, agg_ref, b3_ref, out_ref):
    d = dis_ref[...]
    o = d * (agg_ref[0] + agg_ref[1] + ys3_ref[...]) + b3_ref[...]
    lg = o[:, :10]
    m = jnp.max(lg, axis=1, keepdims=True)
    e = jnp.exp(lg - m)
    res = lg - m - jnp.log(jnp.sum(e, axis=1, keepdims=True))
    out_ref[...] = jnp.concatenate(
        [res, jnp.zeros((res.shape[0], 6), jnp.float32)], axis=1)


def kernel(x, edge_index, W1, b1, W2, b2, W3, b3):
    N, D_IN = x.shape
    E = edge_index.shape[1]
    H1 = W1.shape[1]
    H2 = W2.shape[1]
    C = W3.shape[1]
    CP = 16

    K1 = 32
    K2 = 64
    NPAD = _ceil_to(N + 1, NS * 8)

    src = edge_index[0]
    dst = edge_index[1]

    def _chunked(K, F0):
        tot = -(-E // (NS * K))
        ch0 = max(2, int(round(F0 * tot / 2)) * 2)
        ch1 = max(2, -(-(tot - ch0) // 2) * 2)
        ep = NS * K * (ch0 + ch1)
        s, d = src, dst
        if ep != E:
            pad = ep - E
            s = jnp.concatenate([s, jnp.zeros((pad,), s.dtype)])
            d = jnp.concatenate([d, jnp.full((pad,), N, d.dtype)])
        return ch0, ch1, s.reshape(ep // K, K), d.reshape(ep // K, K)

    def _chunked_full(K):
        tot = -(-E // (NS * K))
        ch = (tot + 1) // 2 * 2
        ep = NS * K * ch
        s, d = src, dst
        if ep != E:
            pad = ep - E
            s = jnp.concatenate([s, jnp.zeros((pad,), s.dtype)])
            d = jnp.concatenate([d, jnp.full((pad,), N, d.dtype)])
        return ch, s.reshape(ep // K, K), d.reshape(ep // K, K)

    AC, src2a, dst2a = _chunked_full(K1)
    B0, B1, src2b, dst2b = _chunked(K2, 0.50)

    W3p = jnp.pad(W3, ((0, 0), (0, CP - C)))
    b1r = b1.reshape(1, H1)
    b2r = b2.reshape(1, H2)
    b3r = jnp.pad(b3, (0, CP - C)).reshape(1, CP)

    BN = 2000
    G = N // BN
    f32 = jnp.float32

    cnt = _sc_deg(NPAD, B0, B1, K2)(dst2b)

    DH = D_IN // 2
    dis, xs = pl.pallas_call(
        _pre_body,
        grid=(G,),
        in_specs=[
            pl.BlockSpec((NC, BN, 16), lambda i: (0, i, 0)),
            pl.BlockSpec((BN, D_IN), lambda i: (i, 0)),
        ],
        out_specs=[
            pl.BlockSpec((BN, 1), lambda i: (i, 0)),
            pl.BlockSpec((NC, BN, DH), lambda i: (0, i, 0)),
        ],
        out_shape=[
            jax.ShapeDtypeStruct((N, 1), f32),
            jax.ShapeDtypeStruct((NC, N, DH), f32),
        ],
    )(cnt, x)

    agg1 = _sc_agg(N, NPAD, AC, AC, DH, K1, 5, LOCAL=True,
                   COLSPLIT=True)(xs, src2a, dst2a)

    ys2 = pl.pallas_call(
        _l1_body,
        grid=(G,),
        in_specs=[
            pl.BlockSpec((BN, 1), lambda i: (i, 0)),
            pl.BlockSpec((NC, BN, DH), lambda i: (0, i, 0)),
            pl.BlockSpec((NC, BN, DH), lambda i: (0, i, 0)),
            pl.BlockSpec((D_IN, H1), lambda i: (0, 0)),
            pl.BlockSpec((1, H1), lambda i: (0, 0)),
            pl.BlockSpec((H1, H2), lambda i: (0, 0)),
        ],
        out_specs=pl.BlockSpec((BN, H2), lambda i: (i, 0)),
        out_shape=jax.ShapeDtypeStruct((N, H2), f32),
    )(dis, xs, agg1, W1, b1r, W2)

    agg2 = _sc_agg(N, NPAD, B0, B1, H2, K2, 6, LOCAL=True)(ys2, src2b, dst2b)

    ys3 = pl.pallas_call(
        _l2_body,
        grid=(G,),
        in_specs=[
            pl.BlockSpec((BN, 1), lambda i: (i, 0)),
            pl.BlockSpec((BN, H2), lambda i: (i, 0)),
            pl.BlockSpec((NC, BN, H2), lambda i: (0, i, 0)),
            pl.BlockSpec((1, H2), lambda i: (0, 0)),
            pl.BlockSpec((H2, CP), lambda i: (0, 0)),
        ],
        out_specs=pl.BlockSpec((BN, CP), lambda i: (i, 0)),
        out_shape=jax.ShapeDtypeStruct((N, CP), f32),
    )(dis, ys2, agg2, b2r, W3p)

    agg3 = _sc_agg(N, NPAD, B0, B1, CP, K2, 6, LOCAL=True)(ys3, src2b, dst2b)

    outp = pl.pallas_call(
        _l3_body,
        grid=(G,),
        in_specs=[
            pl.BlockSpec((BN, 1), lambda i: (i, 0)),
            pl.BlockSpec((BN, CP), lambda i: (i, 0)),
            pl.BlockSpec((NC, BN, CP), lambda i: (0, i, 0)),
            pl.BlockSpec((1, CP), lambda i: (0, 0)),
        ],
        out_specs=pl.BlockSpec((BN, CP), lambda i: (i, 0)),
        out_shape=jax.ShapeDtypeStruct((N, CP), f32),
    )(dis, ys3, agg3, b3r)

    return outp[:, :C]
